# baseline (speedup 1.0000x reference)
import functools

import jax
import jax.numpy as jnp
from jax import lax
from jax.experimental import pallas as pl
from jax.experimental.pallas import tpu as pltpu
from jax.experimental.pallas import tpu_sc as plsc

_B = 4
_T = 4096
_H = 1024
_P = 512
_O = 768
_CW = 128
_NC = 2
_NS = 16
_G = 8
_CG = _H // _G
_CHUNK = 64

_TPH = _T // 2
_NCH = _TPH // _CHUNK
_PH = _P // 2


def _sc_pool(h, pid3, zsum, ones):
    mesh = plsc.VectorSubcoreMesh(core_axis_name="c", subcore_axis_name="s")

    @functools.partial(
        pl.kernel,
        out_type=[
            jax.ShapeDtypeStruct((_B, _G, _P, _CG), jnp.float32),
            jax.ShapeDtypeStruct((_B, _P, _CW), jnp.float32),
        ],
        mesh=mesh,
        scratch_types=[
            pltpu.VMEM((_PH, _CG), jnp.float32),
            pltpu.VMEM((_CHUNK, _CG), jnp.float32),
            pltpu.VMEM((_NCH, _CHUNK), jnp.int32),
            pltpu.VMEM_SHARED((_G, _P, _CG), jnp.float32),
            pltpu.VMEM_SHARED((_G, _P, _CG), jnp.float32),
            pltpu.VMEM_SHARED((_P, _CW), jnp.float32),
            pltpu.VMEM_SHARED((_P, _CW), jnp.float32),
        ],
    )
    def k(h_hbm, pid_hbm, zsum_hbm, ones_hbm, sums_hbm, cnts_hbm,
          zero_v, chunk_v, idx_v, acc0, acc1, cacc0, cacc1):
        c = lax.axis_index("c")
        s = lax.axis_index("s")
        g = s % _G
        th = s // _G
        col0 = g * _CG
        tok0 = th * _TPH

        pltpu.sync_copy(zsum_hbm, zero_v)
        for acc in (acc0, acc1):
            pltpu.sync_copy(zero_v, acc.at[s // 2, pl.ds((s % 2) * _PH, _PH)])
        for i, cacc in enumerate((cacc0, cacc1)):
            @pl.when(s // 2 == i)
            def _():
                pltpu.sync_copy(zero_v, cacc.at[pl.ds((s % 2) * _PH, _PH)])
        plsc.subcore_barrier()

        for bi, (acc, cacc) in enumerate(((acc0, cacc0), (acc1, cacc1))):
            b = c * 2 + bi
            pltpu.sync_copy(pid_hbm.at[b, pl.ds(th * _NCH, _NCH)], idx_v)
            for j in range(_NCH):
                pltpu.sync_copy(
                    h_hbm.at[b, pl.ds(tok0 + j * _CHUNK, _CHUNK),
                             pl.ds(col0, _CG)],
                    chunk_v)
                pltpu.sync_copy(chunk_v, acc.at[g].at[idx_v.at[j]], add=True)

            @pl.when(g == bi)
            def _():
                pltpu.sync_copy(ones_hbm, chunk_v)
                for j in range(_NCH):
                    pltpu.sync_copy(chunk_v, cacc.at[idx_v.at[j]], add=True)
        plsc.subcore_barrier()

        for bi, (acc, cacc) in enumerate(((acc0, cacc0), (acc1, cacc1))):
            b = c * 2 + bi
            pltpu.sync_copy(acc.at[s // 2, pl.ds((s % 2) * _PH, _PH)],
                            sums_hbm.at[b, s // 2, pl.ds((s % 2) * _PH, _PH)])
            @pl.when(s // 2 == bi)
            def _():
                pltpu.sync_copy(cacc.at[pl.ds((s % 2) * _PH, _PH)],
                                cnts_hbm.at[b, pl.ds((s % 2) * _PH, _PH)])

    return k(h, pid3, zsum, ones)


def _tc_project_body(sums_ref, cnts_ref, w_ref, b_ref, out_ref, acc_ref):
    k = pl.program_id(1)
    nk = pl.num_programs(1)

    @pl.when(k == 0)
    def _():
        acc_ref[...] = jnp.zeros_like(acc_ref)

    acc_ref[...] += jnp.dot(sums_ref[0, 0], w_ref[0],
                            preferred_element_type=jnp.float32,
                            precision=lax.Precision.HIGHEST)

    @pl.when(k == nk - 1)
    def _():
        cnt = cnts_ref[0, :, 0:1]
        inv = 1.0 / jnp.maximum(cnt, 1.0)
        out_ref[0] = acc_ref[...] * inv + b_ref[...]


def _tc_project(sums, cnts, w3, b2):
    return pl.pallas_call(
        _tc_project_body,
        grid=(_B, _G),
        in_specs=[
            pl.BlockSpec((1, 1, _P, _CG), lambda b, k: (b, k, 0, 0)),
            pl.BlockSpec((1, _P, _CW), lambda b, k: (b, 0, 0)),
            pl.BlockSpec((1, _CG, _O), lambda b, k: (k, 0, 0)),
            pl.BlockSpec((1, _O), lambda b, k: (0, 0)),
        ],
        out_specs=pl.BlockSpec((1, _P, _O), lambda b, k: (b, 0, 0)),
        out_shape=jax.ShapeDtypeStruct((_B, _P, _O), jnp.float32),
        scratch_shapes=[pltpu.VMEM((_P, _O), jnp.float32)],
    )(sums, cnts, w3, b2)


def kernel(byte_hiddens, patch_ids, W_proj, b_proj):
    pid3 = patch_ids.astype(jnp.int32).reshape(_B, _T // _CHUNK, _CHUNK)
    zsum = jnp.zeros((_PH, _CG), jnp.float32)
    ones = jnp.zeros((_CHUNK, _CW), jnp.float32).at[:, 0].set(1.0)
    sums, cnts = _sc_pool(byte_hiddens, pid3, zsum, ones)
    return _tc_project(sums, cnts, W_proj.reshape(_G, _CG, _O),
                       b_proj.reshape(1, _O))
